# Initial kernel scaffold; baseline (speedup 1.0000x reference)
import jax
import jax.numpy as jnp
from jax.experimental import pallas as pl


def kernel(x_temporal, edge_index, edge_weight, batch, W_xz, b_xz, W_hz, b_hz, W_xr, b_xr, W_hr, b_hr, W_xh, b_xh, W_hh, b_hh, lin1_W, lin1_b, lin2_W, lin2_b):
    raise NotImplementedError("write your pallas kernel here")


# speedup vs baseline: 1.0790x; 1.0790x over previous
import functools

import jax
import jax.numpy as jnp
import numpy as np
from jax import lax
from jax.experimental import pallas as pl
from jax.experimental.pallas import tpu as pltpu

T, N, E = 12, 10000, 160000
EMBED, OUT_CH = 16, 32
NUM_NODES, OUTPUT_SIZE, K = 100, 16, 3
SIZE_IN1 = OUT_CH * NUM_NODES
SIZE_OUT1 = SIZE_IN1 * 2

_ROW_BLK = 1000
_XC_BLK = 2000
_MLP_BLK = 640


def _xc_kernel(f_ref, w_ref, b_ref, o_ref):
    o_ref[...] = f_ref[...] @ w_ref[...] + b_ref[...]


def _xc_matmul(F, W, b):
    M = F.shape[0]
    grid = (M // _XC_BLK,)
    return pl.pallas_call(
        _xc_kernel,
        grid=grid,
        in_specs=[
            pl.BlockSpec((_XC_BLK, 3 * EMBED), lambda i: (i, 0)),
            pl.BlockSpec((3 * EMBED, 3 * OUT_CH), lambda i: (0, 0)),
            pl.BlockSpec((1, 3 * OUT_CH), lambda i: (0, 0)),
        ],
        out_specs=pl.BlockSpec((_XC_BLK, 3 * OUT_CH), lambda i: (i, 0)),
        out_shape=jax.ShapeDtypeStruct((M, 3 * OUT_CH), jnp.float32),
    )(F, W, b)


def _gates_zr_kernel(xc_ref, h_ref, p1_ref, p2_ref, w_ref, b_ref, z_ref, rh_ref):
    h = h_ref[...]
    acc = xc_ref[...][:, : 2 * OUT_CH] + b_ref[...]
    acc = acc + h @ w_ref[0] + p1_ref[...] @ w_ref[1] + p2_ref[...] @ w_ref[2]
    g = jax.nn.sigmoid(acc)
    z_ref[...] = g[:, :OUT_CH]
    rh_ref[...] = g[:, OUT_CH:] * h


def _gates_zr(xc, h, p1, p2, w, b):
    grid = (N // _ROW_BLK,)
    blk = lambda c: pl.BlockSpec((_ROW_BLK, c), lambda i: (i, 0))
    return pl.pallas_call(
        _gates_zr_kernel,
        grid=grid,
        in_specs=[
            blk(3 * OUT_CH), blk(OUT_CH), blk(OUT_CH), blk(OUT_CH),
            pl.BlockSpec((K, OUT_CH, 2 * OUT_CH), lambda i: (0, 0, 0)),
            pl.BlockSpec((1, 2 * OUT_CH), lambda i: (0, 0)),
        ],
        out_specs=[blk(OUT_CH), blk(OUT_CH)],
        out_shape=[
            jax.ShapeDtypeStruct((N, OUT_CH), jnp.float32),
            jax.ShapeDtypeStruct((N, OUT_CH), jnp.float32),
        ],
    )(xc, h, p1, p2, w, b)


def _gates_h_kernel(xc_ref, rh_ref, p1_ref, p2_ref, z_ref, h_ref, w_ref, b_ref, o_ref):
    ht = jnp.tanh(
        xc_ref[...][:, 2 * OUT_CH :]
        + b_ref[...]
        + rh_ref[...] @ w_ref[0]
        + p1_ref[...] @ w_ref[1]
        + p2_ref[...] @ w_ref[2]
    )
    z = z_ref[...]
    o_ref[...] = z * h_ref[...] + (1.0 - z) * ht


def _gates_h(xc, rh, p1, p2, z, h, w, b):
    grid = (N // _ROW_BLK,)
    blk = lambda c: pl.BlockSpec((_ROW_BLK, c), lambda i: (i, 0))
    return pl.pallas_call(
        _gates_h_kernel,
        grid=grid,
        in_specs=[
            blk(3 * OUT_CH), blk(OUT_CH), blk(OUT_CH), blk(OUT_CH),
            blk(OUT_CH), blk(OUT_CH),
            pl.BlockSpec((K, OUT_CH, OUT_CH), lambda i: (0, 0, 0)),
            pl.BlockSpec((1, OUT_CH), lambda i: (0, 0)),
        ],
        out_specs=blk(OUT_CH),
        out_shape=jax.ShapeDtypeStruct((N, OUT_CH), jnp.float32),
    )(xc, rh, p1, p2, z, h, w, b)


def _mlp_kernel(x_ref, w1_ref, b1_ref, w2_ref, b2_ref, o_ref):
    j = pl.program_id(0)
    nb = pl.num_programs(0)

    @pl.when(j == 0)
    def _():
        o_ref[...] = jnp.zeros_like(o_ref)

    x = jnp.maximum(x_ref[...], 0.0)
    h = jnp.maximum(x @ w1_ref[...] + b1_ref[...], 0.0)
    o_ref[...] += h @ w2_ref[...]

    @pl.when(j == nb - 1)
    def _():
        logits = o_ref[...] + b2_ref[...]
        m = jnp.max(logits, axis=-1, keepdims=True)
        e = jnp.exp(logits - m)
        o_ref[...] = e / jnp.sum(e, axis=-1, keepdims=True)


def _mlp_head(x, w1, b1, w2, b2):
    bsz = x.shape[0]
    grid = (SIZE_OUT1 // _MLP_BLK,)
    return pl.pallas_call(
        _mlp_kernel,
        grid=grid,
        in_specs=[
            pl.BlockSpec((bsz, SIZE_IN1), lambda j: (0, 0)),
            pl.BlockSpec((SIZE_IN1, _MLP_BLK), lambda j: (0, j)),
            pl.BlockSpec((1, _MLP_BLK), lambda j: (0, j)),
            pl.BlockSpec((_MLP_BLK, OUTPUT_SIZE), lambda j: (j, 0)),
            pl.BlockSpec((1, OUTPUT_SIZE), lambda j: (0, 0)),
        ],
        out_specs=pl.BlockSpec((bsz, OUTPUT_SIZE), lambda j: (0, 0)),
        out_shape=jax.ShapeDtypeStruct((bsz, OUTPUT_SIZE), jnp.float32),
    )(x, w1, b1, w2, b2)


def kernel(x_temporal, edge_index, edge_weight, batch,
           W_xz, b_xz, W_hz, b_hz, W_xr, b_xr, W_hr, b_hr,
           W_xh, b_xh, W_hh, b_hh, lin1_W, lin1_b, lin2_W, lin2_b):
    row, col = edge_index[0], edge_index[1]
    deg = jnp.zeros((N,), jnp.float32).at[row].add(edge_weight)
    dis = jnp.where(deg > 0, lax.rsqrt(jnp.where(deg > 0, deg, 1.0)), 0.0)
    norm = -dis[row] * edge_weight * dis[col]

    def prop(x):
        return jnp.zeros_like(x).at[col].add(norm[:, None] * x[row])

    X_all = x_temporal.transpose(1, 0, 2).reshape(N, T * EMBED)
    T1 = prop(X_all)
    T2 = 2.0 * prop(T1) - X_all
    F = jnp.concatenate(
        [x_temporal,
         T1.reshape(N, T, EMBED).transpose(1, 0, 2),
         T2.reshape(N, T, EMBED).transpose(1, 0, 2)],
        axis=2,
    ).reshape(T * N, 3 * EMBED)
    W_x = jnp.concatenate(
        [jnp.concatenate([W_xz[k], W_xr[k], W_xh[k]], axis=1) for k in range(K)],
        axis=0,
    )
    b_x = jnp.concatenate([b_xz, b_xr, b_xh]).reshape(1, 3 * OUT_CH)
    XC = _xc_matmul(F, W_x, b_x).reshape(T, N, 3 * OUT_CH)

    W_zr = jnp.stack(
        [jnp.concatenate([W_hz[k], W_hr[k]], axis=1) for k in range(K)]
    )
    b_zr = jnp.concatenate([b_hz, b_hr]).reshape(1, 2 * OUT_CH)
    b_h2 = b_hh.reshape(1, OUT_CH)

    def step(H, xc_t):
        Hp1 = prop(H)
        Hp2 = 2.0 * prop(Hp1) - H
        Z, RH = _gates_zr(xc_t, H, Hp1, Hp2, W_zr, b_zr)
        Rp1 = prop(RH)
        Rp2 = 2.0 * prop(Rp1) - RH
        Hn = _gates_h(xc_t, RH, Rp1, Rp2, Z, H, W_hh, b_h2)
        return Hn, None

    H0 = jnp.zeros((N, OUT_CH), jnp.float32)
    H, _ = lax.scan(step, H0, XC)

    xb = H.reshape(NUM_NODES, SIZE_IN1)
    return _mlp_head(xb, lin1_W, lin1_b.reshape(1, -1), lin2_W,
                     lin2_b.reshape(1, -1))


# speedup vs baseline: 4.9488x; 4.5865x over previous
import functools

import jax
import jax.numpy as jnp
import numpy as np
from jax import lax
from jax.experimental import pallas as pl
from jax.experimental.pallas import tpu as pltpu
from jax.experimental.pallas import tpu_sc as plsc

T, N, E = 12, 10000, 160000
EMBED, OUT_CH = 16, 32
NUM_NODES, OUTPUT_SIZE, K = 100, 16, 3
SIZE_IN1 = OUT_CH * NUM_NODES
SIZE_OUT1 = SIZE_IN1 * 2

_ROW_BLK = 1000
_XC_BLK = 2000
_MLP_BLK = 640

_NSC, _NSUB = 2, 16
_NW = _NSC * _NSUB
_CHUNK = 128
_EPAD = ((E + _NW * _CHUNK - 1) // (_NW * _CHUNK)) * (_NW * _CHUNK)
_NCHUNK = _EPAD // (_NW * _CHUNK)
_NPAD = 10240
_TROWS = _NPAD // _NSUB


def _make_prop_sc(C, nsrc):
    grp = C // 16
    mesh = plsc.VectorSubcoreMesh(
        core_axis_name="c", subcore_axis_name="s",
        num_cores=_NSC, num_subcores=_NSUB)

    def body(*refs):
        (row_hbm, col_hbm, norm_hbm) = refs[:3]
        xs = refs[3:3 + nsrc]
        zeros_hbm = refs[3 + nsrc]
        out0, out1 = refs[4 + nsrc], refs[5 + nsrc]
        sc = refs[6 + nsrc:]
        idxr_v, idxc_v, norm_v = sc[0], sc[1], sc[2]
        rows_v = sc[3]
        if nsrc == 2:
            rows2_v = sc[4]
            acc_sh, sem, sem2 = sc[5], sc[6], sc[7]
        else:
            acc_sh, sem = sc[4], sc[5]

        c = lax.axis_index("c")
        s = lax.axis_index("s")
        w = c * _NSUB + s
        sl = pl.ds(s * _TROWS, _TROWS)

        pltpu.sync_copy(zeros_hbm.at[sl], acc_sh.at[sl])
        pltpu.sync_copy(row_hbm.at[w], idxr_v)
        pltpu.sync_copy(col_hbm.at[w], idxc_v)
        pltpu.sync_copy(norm_hbm.at[w], norm_v)
        plsc.subcore_barrier()

        def chunk(j, carry):
            cp1 = pltpu.async_copy(xs[0].at[idxr_v.at[j]], rows_v, sem)
            if nsrc == 2:
                cp2 = pltpu.async_copy(xs[1].at[idxr_v.at[j]], rows2_v, sem2)
            cp1.wait()
            if nsrc == 2:
                cp2.wait()

            def scale16(eb, carry2):
                nv16 = norm_v[j, pl.ds(eb * 16, 16)]
                for l in range(16):
                    nv = nv16[l]
                    e = eb * 16 + l
                    for g in range(grp):
                        v = rows_v[e, pl.ds(g * 16, 16)]
                        if nsrc == 2:
                            v = v + rows2_v[e, pl.ds(g * 16, 16)]
                        rows_v[e, pl.ds(g * 16, 16)] = v * nv
                return carry2

            lax.fori_loop(0, _CHUNK // 16, scale16, 0)
            pltpu.sync_copy(rows_v, acc_sh.at[idxc_v.at[j]], add=True)
            return carry

        lax.fori_loop(0, _NCHUNK, chunk, 0)
        plsc.subcore_barrier()

        @pl.when(c == 0)
        def _():
            pltpu.sync_copy(acc_sh.at[sl], out0.at[sl])

        @pl.when(c == 1)
        def _():
            pltpu.sync_copy(acc_sh.at[sl], out1.at[sl])

    scratch = [
        pltpu.VMEM((_NCHUNK, _CHUNK), jnp.int32),
        pltpu.VMEM((_NCHUNK, _CHUNK), jnp.int32),
        pltpu.VMEM((_NCHUNK, _CHUNK), jnp.float32),
        pltpu.VMEM((_CHUNK, C), jnp.float32),
    ]
    if nsrc == 2:
        scratch.append(pltpu.VMEM((_CHUNK, C), jnp.float32))
    scratch.append(pltpu.VMEM_SHARED((_NPAD, C), jnp.float32))
    scratch.append(pltpu.SemaphoreType.DMA)
    if nsrc == 2:
        scratch.append(pltpu.SemaphoreType.DMA)
    out_t = [jax.ShapeDtypeStruct((_NPAD, C), jnp.float32)] * 2
    return pl.kernel(body, out_type=out_t, mesh=mesh, scratch_types=scratch,
                     compiler_params=pltpu.CompilerParams(
                         use_tc_tiling_on_sc=False))


def _xc_kernel(f_ref, w_ref, b_ref, o_ref):
    o_ref[...] = f_ref[...] @ w_ref[...] + b_ref[...]


def _xc_matmul(F, W, b):
    M = F.shape[0]
    grid = (M // _XC_BLK,)
    return pl.pallas_call(
        _xc_kernel,
        grid=grid,
        in_specs=[
            pl.BlockSpec((_XC_BLK, 3 * EMBED), lambda i: (i, 0)),
            pl.BlockSpec((3 * EMBED, 3 * OUT_CH), lambda i: (0, 0)),
            pl.BlockSpec((1, 3 * OUT_CH), lambda i: (0, 0)),
        ],
        out_specs=pl.BlockSpec((_XC_BLK, 3 * OUT_CH), lambda i: (i, 0)),
        out_shape=jax.ShapeDtypeStruct((M, 3 * OUT_CH), jnp.float32),
    )(F, W, b)


def _gates_zr_kernel(xc_ref, h_ref, q0_ref, q1_ref, qq0_ref, qq1_ref,
                     w_ref, b_ref, z_ref, rh_ref):
    h = h_ref[...]
    p1 = q0_ref[...] + q1_ref[...]
    p2 = 2.0 * (qq0_ref[...] + qq1_ref[...]) - h
    acc = xc_ref[...][:, : 2 * OUT_CH] + b_ref[...]
    acc = acc + h @ w_ref[0] + p1 @ w_ref[1] + p2 @ w_ref[2]
    g = jax.nn.sigmoid(acc)
    z_ref[...] = g[:, :OUT_CH]
    rh_ref[...] = g[:, OUT_CH:] * h


def _gates_zr(xc, h, q0, q1, qq0, qq1, w, b):
    grid = (N // _ROW_BLK,)
    blk = lambda c: pl.BlockSpec((_ROW_BLK, c), lambda i: (i, 0))
    return pl.pallas_call(
        _gates_zr_kernel,
        grid=grid,
        in_specs=[
            blk(3 * OUT_CH), blk(OUT_CH), blk(OUT_CH), blk(OUT_CH),
            blk(OUT_CH), blk(OUT_CH),
            pl.BlockSpec((K, OUT_CH, 2 * OUT_CH), lambda i: (0, 0, 0)),
            pl.BlockSpec((1, 2 * OUT_CH), lambda i: (0, 0)),
        ],
        out_specs=[blk(OUT_CH), blk(OUT_CH)],
        out_shape=[
            jax.ShapeDtypeStruct((N, OUT_CH), jnp.float32),
            jax.ShapeDtypeStruct((N, OUT_CH), jnp.float32),
        ],
    )(xc, h, q0, q1, qq0, qq1, w, b)


def _gates_h_kernel(xc_ref, rh_ref, q0_ref, q1_ref, qq0_ref, qq1_ref,
                    z_ref, h_ref, w_ref, b_ref, o_ref):
    rh = rh_ref[...]
    p1 = q0_ref[...] + q1_ref[...]
    p2 = 2.0 * (qq0_ref[...] + qq1_ref[...]) - rh
    ht = jnp.tanh(
        xc_ref[...][:, 2 * OUT_CH :]
        + b_ref[...]
        + rh @ w_ref[0]
        + p1 @ w_ref[1]
        + p2 @ w_ref[2]
    )
    z = z_ref[...]
    o_ref[...] = z * h_ref[...] + (1.0 - z) * ht


def _gates_h(xc, rh, q0, q1, qq0, qq1, z, h, w, b):
    grid = (N // _ROW_BLK,)
    blk = lambda c: pl.BlockSpec((_ROW_BLK, c), lambda i: (i, 0))
    return pl.pallas_call(
        _gates_h_kernel,
        grid=grid,
        in_specs=[
            blk(3 * OUT_CH), blk(OUT_CH), blk(OUT_CH), blk(OUT_CH),
            blk(OUT_CH), blk(OUT_CH), blk(OUT_CH), blk(OUT_CH),
            pl.BlockSpec((K, OUT_CH, OUT_CH), lambda i: (0, 0, 0)),
            pl.BlockSpec((1, OUT_CH), lambda i: (0, 0)),
        ],
        out_specs=blk(OUT_CH),
        out_shape=jax.ShapeDtypeStruct((N, OUT_CH), jnp.float32),
    )(xc, rh, q0, q1, qq0, qq1, z, h, w, b)


def _mlp_kernel(x_ref, w1_ref, b1_ref, w2_ref, b2_ref, o_ref):
    j = pl.program_id(0)
    nb = pl.num_programs(0)

    @pl.when(j == 0)
    def _():
        o_ref[...] = jnp.zeros_like(o_ref)

    x = jnp.maximum(x_ref[...], 0.0)
    h = jnp.maximum(x @ w1_ref[...] + b1_ref[...], 0.0)
    o_ref[...] += h @ w2_ref[...]

    @pl.when(j == nb - 1)
    def _():
        logits = o_ref[...] + b2_ref[...]
        m = jnp.max(logits, axis=-1, keepdims=True)
        e = jnp.exp(logits - m)
        o_ref[...] = e / jnp.sum(e, axis=-1, keepdims=True)


def _mlp_head(x, w1, b1, w2, b2):
    bsz = x.shape[0]
    grid = (SIZE_OUT1 // _MLP_BLK,)
    return pl.pallas_call(
        _mlp_kernel,
        grid=grid,
        in_specs=[
            pl.BlockSpec((bsz, SIZE_IN1), lambda j: (0, 0)),
            pl.BlockSpec((SIZE_IN1, _MLP_BLK), lambda j: (0, j)),
            pl.BlockSpec((1, _MLP_BLK), lambda j: (0, j)),
            pl.BlockSpec((_MLP_BLK, OUTPUT_SIZE), lambda j: (j, 0)),
            pl.BlockSpec((1, OUTPUT_SIZE), lambda j: (0, 0)),
        ],
        out_specs=pl.BlockSpec((bsz, OUTPUT_SIZE), lambda j: (0, 0)),
        out_shape=jax.ShapeDtypeStruct((bsz, OUTPUT_SIZE), jnp.float32),
    )(x, w1, b1, w2, b2)


def kernel(x_temporal, edge_index, edge_weight, batch,
           W_xz, b_xz, W_hz, b_hz, W_xr, b_xr, W_hr, b_hr,
           W_xh, b_xh, W_hh, b_hh, lin1_W, lin1_b, lin2_W, lin2_b):
    row, col = edge_index[0], edge_index[1]
    deg = jnp.zeros((N,), jnp.float32).at[row].add(edge_weight)
    dis = jnp.where(deg > 0, lax.rsqrt(jnp.where(deg > 0, deg, 1.0)), 0.0)
    norm = -dis[row] * edge_weight * dis[col]

    pad = _EPAD - E
    rowp = jnp.concatenate([row, jnp.zeros((pad,), jnp.int32)]).reshape(
        _NW, _NCHUNK, _CHUNK)
    colp = jnp.concatenate([col, jnp.zeros((pad,), jnp.int32)]).reshape(
        _NW, _NCHUNK, _CHUNK)
    normp = jnp.concatenate([norm, jnp.zeros((pad,), jnp.float32)]).reshape(
        _NW, _NCHUNK, _CHUNK)

    CX = T * EMBED
    z32 = jnp.zeros((_NPAD, OUT_CH), jnp.float32)
    zX = jnp.zeros((_NPAD, CX // 2), jnp.float32)

    CH = CX // 2
    prop32_1 = _make_prop_sc(OUT_CH, 1)
    prop32_2 = _make_prop_sc(OUT_CH, 2)
    propX_1 = _make_prop_sc(CH, 1)
    propX_2 = _make_prop_sc(CH, 2)

    X_all = x_temporal.transpose(1, 0, 2).reshape(N, CX)
    t1h, t2h = [], []
    for XH in (X_all[:, :CH], X_all[:, CH:]):
        a0, a1 = propX_1(rowp, colp, normp, XH, zX)
        b0, b1 = propX_2(rowp, colp, normp, a0, a1, zX)
        t1h.append((a0 + a1)[:N])
        t2h.append(2.0 * (b0 + b1)[:N] - XH)
    T1 = jnp.concatenate(t1h, axis=1)
    T2 = jnp.concatenate(t2h, axis=1)
    F = jnp.concatenate(
        [x_temporal,
         T1.reshape(N, T, EMBED).transpose(1, 0, 2),
         T2.reshape(N, T, EMBED).transpose(1, 0, 2)],
        axis=2,
    ).reshape(T * N, 3 * EMBED)
    W_x = jnp.concatenate(
        [jnp.concatenate([W_xz[k], W_xr[k], W_xh[k]], axis=1) for k in range(K)],
        axis=0,
    )
    b_x = jnp.concatenate([b_xz, b_xr, b_xh]).reshape(1, 3 * OUT_CH)
    XC = _xc_matmul(F, W_x, b_x).reshape(T, N, 3 * OUT_CH)

    W_zr = jnp.stack(
        [jnp.concatenate([W_hz[k], W_hr[k]], axis=1) for k in range(K)]
    )
    b_zr = jnp.concatenate([b_hz, b_hr]).reshape(1, 2 * OUT_CH)
    b_h2 = b_hh.reshape(1, OUT_CH)

    def step(H, xc_t):
        q0, q1 = prop32_1(rowp, colp, normp, H, z32)
        qq0, qq1 = prop32_2(rowp, colp, normp, q0, q1, z32)
        Z, RH = _gates_zr(xc_t, H, q0, q1, qq0, qq1, W_zr, b_zr)
        r0, r1 = prop32_1(rowp, colp, normp, RH, z32)
        rr0, rr1 = prop32_2(rowp, colp, normp, r0, r1, z32)
        Hn = _gates_h(xc_t, RH, r0, r1, rr0, rr1, Z, H, W_hh, b_h2)
        return Hn, None

    H0 = jnp.zeros((N, OUT_CH), jnp.float32)
    H, _ = lax.scan(step, H0, XC)

    xb = H.reshape(NUM_NODES, SIZE_IN1)
    return _mlp_head(xb, lin1_W, lin1_b.reshape(1, -1), lin2_W,
                     lin2_b.reshape(1, -1))


# speedup vs baseline: 5.9676x; 1.2059x over previous
import functools

import jax
import jax.numpy as jnp
import numpy as np
from jax import lax
from jax.experimental import pallas as pl
from jax.experimental.pallas import tpu as pltpu
from jax.experimental.pallas import tpu_sc as plsc

T, N, E = 12, 10000, 160000
EMBED, OUT_CH = 16, 32
NUM_NODES, OUTPUT_SIZE, K = 100, 16, 3
SIZE_IN1 = OUT_CH * NUM_NODES
SIZE_OUT1 = SIZE_IN1 * 2

_ROW_BLK = 1000
_XC_BLK = 2000
_MLP_BLK = 640

_NSC, _NSUB = 2, 16
_NW = _NSC * _NSUB
_CHUNK = 128
_EPAD = ((E + _NW * _CHUNK - 1) // (_NW * _CHUNK)) * (_NW * _CHUNK)
_NCHUNK = _EPAD // (_NW * _CHUNK)
_NPAD = 10240
_TROWS = _NPAD // _NSUB


def _make_prop_sc(C, nsrc, chunk=_CHUNK):
    grp = C // 16
    nchunk = _EPAD // (_NW * chunk)
    mesh = plsc.VectorSubcoreMesh(
        core_axis_name="c", subcore_axis_name="s",
        num_cores=_NSC, num_subcores=_NSUB)

    def body(*refs):
        (row_hbm, col_hbm, norm_hbm) = refs[:3]
        xs = refs[3:3 + nsrc]
        zeros_hbm = refs[3 + nsrc]
        out0, out1 = refs[4 + nsrc], refs[5 + nsrc]
        sc = refs[6 + nsrc:]
        idxr_v, idxc_v, norm_v = sc[0], sc[1], sc[2]
        gA = sc[3:5]
        sbuf = sc[5:7]
        if nsrc == 2:
            gB = sc[7:9]
            acc_sh = sc[9]
            semga = sc[10:12]
            semsc = sc[12:14]
            semgb = sc[14:16]
        else:
            acc_sh = sc[7]
            semga = sc[8:10]
            semsc = sc[10:12]

        c = lax.axis_index("c")
        s_ = lax.axis_index("s")
        w = c * _NSUB + s_
        sl = pl.ds(s_ * _TROWS, _TROWS)

        pltpu.sync_copy(zeros_hbm.at[sl], acc_sh.at[sl])
        pltpu.sync_copy(row_hbm.at[w], idxr_v)
        pltpu.sync_copy(col_hbm.at[w], idxc_v)
        pltpu.sync_copy(norm_hbm.at[w], norm_v)
        plsc.subcore_barrier()

        def issue_gather(j, b):
            pltpu.async_copy(xs[0].at[idxr_v.at[j]], gA[b], semga[b])
            if nsrc == 2:
                pltpu.async_copy(xs[1].at[idxr_v.at[j]], gB[b], semgb[b])

        def wait_gather(j, b):
            pltpu.make_async_copy(xs[0].at[idxr_v.at[j]], gA[b], semga[b]).wait()
            if nsrc == 2:
                pltpu.make_async_copy(xs[1].at[idxr_v.at[j]], gB[b], semgb[b]).wait()

        def do_scale(j, b):
            def scale16(eb, carry):
                nv16 = norm_v[j, pl.ds(eb * 16, 16)]
                for l in range(16):
                    nv = nv16[l]
                    e = eb * 16 + l
                    for g in range(grp):
                        v = gA[b][e, pl.ds(g * 16, 16)]
                        if nsrc == 2:
                            v = v + gB[b][e, pl.ds(g * 16, 16)]
                        sbuf[b][e, pl.ds(g * 16, 16)] = v * nv
                return carry

            lax.fori_loop(0, chunk // 16, scale16, 0)

        def issue_scatter(j, b):
            pltpu.async_copy(sbuf[b], acc_sh.at[idxc_v.at[j]], semsc[b], add=True)

        def wait_scatter(j, b):
            pltpu.make_async_copy(sbuf[b], acc_sh.at[idxc_v.at[j]], semsc[b]).wait()

        for b in range(2):
            issue_gather(b, b)
        for b in range(2):
            wait_gather(b, b)
            do_scale(b, b)
            issue_scatter(b, b)
            issue_gather(b + 2, b)

        def main(jj, carry):
            for b in range(2):
                j = jj * 2 + b
                wait_gather(j, b)
                wait_scatter(j - 2, b)
                do_scale(j, b)
                issue_scatter(j, b)
                issue_gather(j + 2, b)
            return carry

        lax.fori_loop(1, nchunk // 2 - 1, main, 0)

        for b in range(2):
            j = nchunk - 2 + b
            wait_gather(j, b)
            wait_scatter(j - 2, b)
            do_scale(j, b)
            issue_scatter(j, b)
        for b in range(2):
            wait_scatter(nchunk - 2 + b, b)

        plsc.subcore_barrier()

        @pl.when(c == 0)
        def _():
            pltpu.sync_copy(acc_sh.at[sl], out0.at[sl])

        @pl.when(c == 1)
        def _():
            pltpu.sync_copy(acc_sh.at[sl], out1.at[sl])

    scratch = [
        pltpu.VMEM((nchunk, chunk), jnp.int32),
        pltpu.VMEM((nchunk, chunk), jnp.int32),
        pltpu.VMEM((nchunk, chunk), jnp.float32),
        pltpu.VMEM((chunk, C), jnp.float32),
        pltpu.VMEM((chunk, C), jnp.float32),
        pltpu.VMEM((chunk, C), jnp.float32),
        pltpu.VMEM((chunk, C), jnp.float32),
    ]
    if nsrc == 2:
        scratch += [pltpu.VMEM((chunk, C), jnp.float32),
                    pltpu.VMEM((chunk, C), jnp.float32)]
    scratch.append(pltpu.VMEM_SHARED((_NPAD, C), jnp.float32))
    scratch += [pltpu.SemaphoreType.DMA] * (4 if nsrc == 1 else 6)
    out_t = [jax.ShapeDtypeStruct((_NPAD, C), jnp.float32)] * 2
    return pl.kernel(body, out_type=out_t, mesh=mesh, scratch_types=scratch,
                     compiler_params=pltpu.CompilerParams(
                         use_tc_tiling_on_sc=False))


def _xc_kernel(f_ref, w_ref, b_ref, o_ref):
    o_ref[...] = f_ref[...] @ w_ref[...] + b_ref[...]


def _xc_matmul(F, W, b):
    M = F.shape[0]
    grid = (M // _XC_BLK,)
    return pl.pallas_call(
        _xc_kernel,
        grid=grid,
        in_specs=[
            pl.BlockSpec((_XC_BLK, 3 * EMBED), lambda i: (i, 0)),
            pl.BlockSpec((3 * EMBED, 3 * OUT_CH), lambda i: (0, 0)),
            pl.BlockSpec((1, 3 * OUT_CH), lambda i: (0, 0)),
        ],
        out_specs=pl.BlockSpec((_XC_BLK, 3 * OUT_CH), lambda i: (i, 0)),
        out_shape=jax.ShapeDtypeStruct((M, 3 * OUT_CH), jnp.float32),
    )(F, W, b)


def _gates_zr_kernel(xc_ref, h_ref, q0_ref, q1_ref, qq0_ref, qq1_ref,
                     w_ref, b_ref, z_ref, rh_ref):
    h = h_ref[...]
    p1 = q0_ref[...] + q1_ref[...]
    p2 = 2.0 * (qq0_ref[...] + qq1_ref[...]) - h
    acc = xc_ref[...][:, : 2 * OUT_CH] + b_ref[...]
    acc = acc + h @ w_ref[0] + p1 @ w_ref[1] + p2 @ w_ref[2]
    g = jax.nn.sigmoid(acc)
    z_ref[...] = g[:, :OUT_CH]
    rh_ref[...] = g[:, OUT_CH:] * h


def _gates_zr(xc, h, q0, q1, qq0, qq1, w, b):
    grid = (N // _ROW_BLK,)
    blk = lambda c: pl.BlockSpec((_ROW_BLK, c), lambda i: (i, 0))
    return pl.pallas_call(
        _gates_zr_kernel,
        grid=grid,
        in_specs=[
            blk(3 * OUT_CH), blk(OUT_CH), blk(OUT_CH), blk(OUT_CH),
            blk(OUT_CH), blk(OUT_CH),
            pl.BlockSpec((K, OUT_CH, 2 * OUT_CH), lambda i: (0, 0, 0)),
            pl.BlockSpec((1, 2 * OUT_CH), lambda i: (0, 0)),
        ],
        out_specs=[blk(OUT_CH), blk(OUT_CH)],
        out_shape=[
            jax.ShapeDtypeStruct((N, OUT_CH), jnp.float32),
            jax.ShapeDtypeStruct((N, OUT_CH), jnp.float32),
        ],
    )(xc, h, q0, q1, qq0, qq1, w, b)


def _gates_h_kernel(xc_ref, rh_ref, q0_ref, q1_ref, qq0_ref, qq1_ref,
                    z_ref, h_ref, w_ref, b_ref, o_ref):
    rh = rh_ref[...]
    p1 = q0_ref[...] + q1_ref[...]
    p2 = 2.0 * (qq0_ref[...] + qq1_ref[...]) - rh
    ht = jnp.tanh(
        xc_ref[...][:, 2 * OUT_CH :]
        + b_ref[...]
        + rh @ w_ref[0]
        + p1 @ w_ref[1]
        + p2 @ w_ref[2]
    )
    z = z_ref[...]
    o_ref[...] = z * h_ref[...] + (1.0 - z) * ht


def _gates_h(xc, rh, q0, q1, qq0, qq1, z, h, w, b):
    grid = (N // _ROW_BLK,)
    blk = lambda c: pl.BlockSpec((_ROW_BLK, c), lambda i: (i, 0))
    return pl.pallas_call(
        _gates_h_kernel,
        grid=grid,
        in_specs=[
            blk(3 * OUT_CH), blk(OUT_CH), blk(OUT_CH), blk(OUT_CH),
            blk(OUT_CH), blk(OUT_CH), blk(OUT_CH), blk(OUT_CH),
            pl.BlockSpec((K, OUT_CH, OUT_CH), lambda i: (0, 0, 0)),
            pl.BlockSpec((1, OUT_CH), lambda i: (0, 0)),
        ],
        out_specs=blk(OUT_CH),
        out_shape=jax.ShapeDtypeStruct((N, OUT_CH), jnp.float32),
    )(xc, rh, q0, q1, qq0, qq1, z, h, w, b)


def _mlp_kernel(x_ref, w1_ref, b1_ref, w2_ref, b2_ref, o_ref):
    j = pl.program_id(0)
    nb = pl.num_programs(0)

    @pl.when(j == 0)
    def _():
        o_ref[...] = jnp.zeros_like(o_ref)

    x = jnp.maximum(x_ref[...], 0.0)
    h = jnp.maximum(x @ w1_ref[...] + b1_ref[...], 0.0)
    o_ref[...] += h @ w2_ref[...]

    @pl.when(j == nb - 1)
    def _():
        logits = o_ref[...] + b2_ref[...]
        m = jnp.max(logits, axis=-1, keepdims=True)
        e = jnp.exp(logits - m)
        o_ref[...] = e / jnp.sum(e, axis=-1, keepdims=True)


def _mlp_head(x, w1, b1, w2, b2):
    bsz = x.shape[0]
    grid = (SIZE_OUT1 // _MLP_BLK,)
    return pl.pallas_call(
        _mlp_kernel,
        grid=grid,
        in_specs=[
            pl.BlockSpec((bsz, SIZE_IN1), lambda j: (0, 0)),
            pl.BlockSpec((SIZE_IN1, _MLP_BLK), lambda j: (0, j)),
            pl.BlockSpec((1, _MLP_BLK), lambda j: (0, j)),
            pl.BlockSpec((_MLP_BLK, OUTPUT_SIZE), lambda j: (j, 0)),
            pl.BlockSpec((1, OUTPUT_SIZE), lambda j: (0, 0)),
        ],
        out_specs=pl.BlockSpec((bsz, OUTPUT_SIZE), lambda j: (0, 0)),
        out_shape=jax.ShapeDtypeStruct((bsz, OUTPUT_SIZE), jnp.float32),
    )(x, w1, b1, w2, b2)


def kernel(x_temporal, edge_index, edge_weight, batch,
           W_xz, b_xz, W_hz, b_hz, W_xr, b_xr, W_hr, b_hr,
           W_xh, b_xh, W_hh, b_hh, lin1_W, lin1_b, lin2_W, lin2_b):
    row, col = edge_index[0], edge_index[1]
    deg = jnp.zeros((N,), jnp.float32).at[row].add(edge_weight)
    dis = jnp.where(deg > 0, lax.rsqrt(jnp.where(deg > 0, deg, 1.0)), 0.0)
    norm = -dis[row] * edge_weight * dis[col]

    pad = _EPAD - E
    rowp = jnp.concatenate([row, jnp.zeros((pad,), jnp.int32)]).reshape(
        _NW, _NCHUNK, _CHUNK)
    colp = jnp.concatenate([col, jnp.zeros((pad,), jnp.int32)]).reshape(
        _NW, _NCHUNK, _CHUNK)
    normp = jnp.concatenate([norm, jnp.zeros((pad,), jnp.float32)]).reshape(
        _NW, _NCHUNK, _CHUNK)

    CX = T * EMBED
    z32 = jnp.zeros((_NPAD, OUT_CH), jnp.float32)
    zX = jnp.zeros((_NPAD, CX // 2), jnp.float32)

    CH = CX // 2
    prop32_1 = _make_prop_sc(OUT_CH, 1)
    prop32_2 = _make_prop_sc(OUT_CH, 2)
    propX_1 = _make_prop_sc(CH, 1, chunk=64)
    propX_2 = _make_prop_sc(CH, 2, chunk=64)
    rowp64 = rowp.reshape(_NW, -1, 64)
    colp64 = colp.reshape(_NW, -1, 64)
    normp64 = normp.reshape(_NW, -1, 64)

    X_all = x_temporal.transpose(1, 0, 2).reshape(N, CX)
    t1h, t2h = [], []
    for XH in (X_all[:, :CH], X_all[:, CH:]):
        a0, a1 = propX_1(rowp64, colp64, normp64, XH, zX)
        b0, b1 = propX_2(rowp64, colp64, normp64, a0, a1, zX)
        t1h.append((a0 + a1)[:N])
        t2h.append(2.0 * (b0 + b1)[:N] - XH)
    T1 = jnp.concatenate(t1h, axis=1)
    T2 = jnp.concatenate(t2h, axis=1)
    F = jnp.concatenate(
        [x_temporal,
         T1.reshape(N, T, EMBED).transpose(1, 0, 2),
         T2.reshape(N, T, EMBED).transpose(1, 0, 2)],
        axis=2,
    ).reshape(T * N, 3 * EMBED)
    W_x = jnp.concatenate(
        [jnp.concatenate([W_xz[k], W_xr[k], W_xh[k]], axis=1) for k in range(K)],
        axis=0,
    )
    b_x = jnp.concatenate([b_xz, b_xr, b_xh]).reshape(1, 3 * OUT_CH)
    XC = _xc_matmul(F, W_x, b_x).reshape(T, N, 3 * OUT_CH)

    W_zr = jnp.stack(
        [jnp.concatenate([W_hz[k], W_hr[k]], axis=1) for k in range(K)]
    )
    b_zr = jnp.concatenate([b_hz, b_hr]).reshape(1, 2 * OUT_CH)
    b_h2 = b_hh.reshape(1, OUT_CH)

    def step(H, xc_t):
        q0, q1 = prop32_1(rowp, colp, normp, H, z32)
        qq0, qq1 = prop32_2(rowp, colp, normp, q0, q1, z32)
        Z, RH = _gates_zr(xc_t, H, q0, q1, qq0, qq1, W_zr, b_zr)
        r0, r1 = prop32_1(rowp, colp, normp, RH, z32)
        rr0, rr1 = prop32_2(rowp, colp, normp, r0, r1, z32)
        Hn = _gates_h(xc_t, RH, r0, r1, rr0, rr1, Z, H, W_hh, b_h2)
        return Hn, None

    H0 = jnp.zeros((N, OUT_CH), jnp.float32)
    H, _ = lax.scan(step, H0, XC)

    xb = H.reshape(NUM_NODES, SIZE_IN1)
    return _mlp_head(xb, lin1_W, lin1_b.reshape(1, -1), lin2_W,
                     lin2_b.reshape(1, -1))


# speedup vs baseline: 5.9936x; 1.0044x over previous
import functools

import jax
import jax.numpy as jnp
import numpy as np
from jax import lax
from jax.experimental import pallas as pl
from jax.experimental.pallas import tpu as pltpu
from jax.experimental.pallas import tpu_sc as plsc

T, N, E = 12, 10000, 160000
EMBED, OUT_CH = 16, 32
NUM_NODES, OUTPUT_SIZE, K = 100, 16, 3
SIZE_IN1 = OUT_CH * NUM_NODES
SIZE_OUT1 = SIZE_IN1 * 2

_ROW_BLK = 1000
_XC_BLK = 2000
_MLP_BLK = 640

_NSC, _NSUB = 2, 16
_NW = _NSC * _NSUB
_CHUNK = 128
_EPAD = ((E + _NW * _CHUNK - 1) // (_NW * _CHUNK)) * (_NW * _CHUNK)
_NCHUNK = _EPAD // (_NW * _CHUNK)
_NPAD = 10240
_TROWS = _NPAD // _NSUB


def _make_prop_sc(C, nsrc, chunk=_CHUNK):
    grp = C // 16
    nchunk = _EPAD // (_NW * chunk)
    mesh = plsc.VectorSubcoreMesh(
        core_axis_name="c", subcore_axis_name="s",
        num_cores=_NSC, num_subcores=_NSUB)

    def body(*refs):
        (row_hbm, col_hbm, norm_hbm) = refs[:3]
        xs = refs[3:3 + nsrc]
        zeros_hbm = refs[3 + nsrc]
        out0, out1 = refs[4 + nsrc], refs[5 + nsrc]
        sc = refs[6 + nsrc:]
        idxr_v, idxc_v, norm_v = sc[0], sc[1], sc[2]
        gA = sc[3:5]
        sbuf = sc[5:7]
        if nsrc == 2:
            gB = sc[7:9]
            acc_sh = sc[9]
            semga = sc[10:12]
            semsc = sc[12:14]
            semgb = sc[14:16]
        else:
            acc_sh = sc[7]
            semga = sc[8:10]
            semsc = sc[10:12]

        c = lax.axis_index("c")
        s_ = lax.axis_index("s")
        w = c * _NSUB + s_
        sl = pl.ds(s_ * _TROWS, _TROWS)

        pltpu.sync_copy(zeros_hbm.at[sl], acc_sh.at[sl])
        pltpu.sync_copy(row_hbm.at[w], idxr_v)
        pltpu.sync_copy(col_hbm.at[w], idxc_v)
        pltpu.sync_copy(norm_hbm.at[w], norm_v)
        plsc.subcore_barrier()

        def issue_gather(j, b):
            pltpu.async_copy(xs[0].at[idxr_v.at[j]], gA[b], semga[b])
            if nsrc == 2:
                pltpu.async_copy(xs[1].at[idxr_v.at[j]], gB[b], semgb[b])

        def wait_gather(j, b):
            pltpu.make_async_copy(xs[0].at[idxr_v.at[j]], gA[b], semga[b]).wait()
            if nsrc == 2:
                pltpu.make_async_copy(xs[1].at[idxr_v.at[j]], gB[b], semgb[b]).wait()

        def do_scale(j, b):
            for eb in range(chunk // 16):
                nv16 = norm_v[j, pl.ds(eb * 16, 16)]
                for l in range(16):
                    nv = nv16[l]
                    e = eb * 16 + l
                    for g in range(grp):
                        v = gA[b][e, pl.ds(g * 16, 16)]
                        if nsrc == 2:
                            v = v + gB[b][e, pl.ds(g * 16, 16)]
                        sbuf[b][e, pl.ds(g * 16, 16)] = v * nv

        def issue_scatter(j, b):
            pltpu.async_copy(sbuf[b], acc_sh.at[idxc_v.at[j]], semsc[b], add=True)

        def wait_scatter(j, b):
            pltpu.make_async_copy(sbuf[b], acc_sh.at[idxc_v.at[j]], semsc[b]).wait()

        for b in range(2):
            issue_gather(b, b)

        def main(jj, carry):
            for b in range(2):
                j = jj * 2 + b
                wait_gather(j, b)

                @pl.when(jj >= 1)
                def _():
                    wait_scatter(j - 2, b)

                do_scale(j, b)
                issue_scatter(j, b)

                @pl.when(jj < nchunk // 2 - 1)
                def _():
                    issue_gather(j + 2, b)
            return carry

        lax.fori_loop(0, nchunk // 2, main, 0)
        for b in range(2):
            wait_scatter(nchunk - 2 + b, b)

        plsc.subcore_barrier()

        @pl.when(c == 0)
        def _():
            pltpu.sync_copy(acc_sh.at[sl], out0.at[sl])

        @pl.when(c == 1)
        def _():
            pltpu.sync_copy(acc_sh.at[sl], out1.at[sl])

    scratch = [
        pltpu.VMEM((nchunk, chunk), jnp.int32),
        pltpu.VMEM((nchunk, chunk), jnp.int32),
        pltpu.VMEM((nchunk, chunk), jnp.float32),
        pltpu.VMEM((chunk, C), jnp.float32),
        pltpu.VMEM((chunk, C), jnp.float32),
        pltpu.VMEM((chunk, C), jnp.float32),
        pltpu.VMEM((chunk, C), jnp.float32),
    ]
    if nsrc == 2:
        scratch += [pltpu.VMEM((chunk, C), jnp.float32),
                    pltpu.VMEM((chunk, C), jnp.float32)]
    scratch.append(pltpu.VMEM_SHARED((_NPAD, C), jnp.float32))
    scratch += [pltpu.SemaphoreType.DMA] * (4 if nsrc == 1 else 6)
    out_t = [jax.ShapeDtypeStruct((_NPAD, C), jnp.float32)] * 2
    return pl.kernel(body, out_type=out_t, mesh=mesh, scratch_types=scratch,
                     compiler_params=pltpu.CompilerParams(
                         use_tc_tiling_on_sc=False))


def _xc_kernel(f_ref, w_ref, b_ref, o_ref):
    o_ref[...] = f_ref[...] @ w_ref[...] + b_ref[...]


def _xc_matmul(F, W, b):
    M = F.shape[0]
    grid = (M // _XC_BLK,)
    return pl.pallas_call(
        _xc_kernel,
        grid=grid,
        in_specs=[
            pl.BlockSpec((_XC_BLK, 3 * EMBED), lambda i: (i, 0)),
            pl.BlockSpec((3 * EMBED, 3 * OUT_CH), lambda i: (0, 0)),
            pl.BlockSpec((1, 3 * OUT_CH), lambda i: (0, 0)),
        ],
        out_specs=pl.BlockSpec((_XC_BLK, 3 * OUT_CH), lambda i: (i, 0)),
        out_shape=jax.ShapeDtypeStruct((M, 3 * OUT_CH), jnp.float32),
    )(F, W, b)


def _gates_zr_kernel(xc_ref, h_ref, q0_ref, q1_ref, qq0_ref, qq1_ref,
                     w_ref, b_ref, z_ref, rh_ref):
    h = h_ref[...]
    p1 = q0_ref[...] + q1_ref[...]
    p2 = 2.0 * (qq0_ref[...] + qq1_ref[...]) - h
    acc = xc_ref[...][:, : 2 * OUT_CH] + b_ref[...]
    acc = acc + h @ w_ref[0] + p1 @ w_ref[1] + p2 @ w_ref[2]
    g = jax.nn.sigmoid(acc)
    z_ref[...] = g[:, :OUT_CH]
    rh_ref[...] = g[:, OUT_CH:] * h


def _gates_zr(xc, h, q0, q1, qq0, qq1, w, b):
    grid = (N // _ROW_BLK,)
    blk = lambda c: pl.BlockSpec((_ROW_BLK, c), lambda i: (i, 0))
    return pl.pallas_call(
        _gates_zr_kernel,
        grid=grid,
        in_specs=[
            blk(3 * OUT_CH), blk(OUT_CH), blk(OUT_CH), blk(OUT_CH),
            blk(OUT_CH), blk(OUT_CH),
            pl.BlockSpec((K, OUT_CH, 2 * OUT_CH), lambda i: (0, 0, 0)),
            pl.BlockSpec((1, 2 * OUT_CH), lambda i: (0, 0)),
        ],
        out_specs=[blk(OUT_CH), blk(OUT_CH)],
        out_shape=[
            jax.ShapeDtypeStruct((N, OUT_CH), jnp.float32),
            jax.ShapeDtypeStruct((N, OUT_CH), jnp.float32),
        ],
    )(xc, h, q0, q1, qq0, qq1, w, b)


def _gates_h_kernel(xc_ref, rh_ref, q0_ref, q1_ref, qq0_ref, qq1_ref,
                    z_ref, h_ref, w_ref, b_ref, o_ref):
    rh = rh_ref[...]
    p1 = q0_ref[...] + q1_ref[...]
    p2 = 2.0 * (qq0_ref[...] + qq1_ref[...]) - rh
    ht = jnp.tanh(
        xc_ref[...][:, 2 * OUT_CH :]
        + b_ref[...]
        + rh @ w_ref[0]
        + p1 @ w_ref[1]
        + p2 @ w_ref[2]
    )
    z = z_ref[...]
    o_ref[...] = z * h_ref[...] + (1.0 - z) * ht


def _gates_h(xc, rh, q0, q1, qq0, qq1, z, h, w, b):
    grid = (N // _ROW_BLK,)
    blk = lambda c: pl.BlockSpec((_ROW_BLK, c), lambda i: (i, 0))
    return pl.pallas_call(
        _gates_h_kernel,
        grid=grid,
        in_specs=[
            blk(3 * OUT_CH), blk(OUT_CH), blk(OUT_CH), blk(OUT_CH),
            blk(OUT_CH), blk(OUT_CH), blk(OUT_CH), blk(OUT_CH),
            pl.BlockSpec((K, OUT_CH, OUT_CH), lambda i: (0, 0, 0)),
            pl.BlockSpec((1, OUT_CH), lambda i: (0, 0)),
        ],
        out_specs=blk(OUT_CH),
        out_shape=jax.ShapeDtypeStruct((N, OUT_CH), jnp.float32),
    )(xc, rh, q0, q1, qq0, qq1, z, h, w, b)


def _mlp_kernel(x_ref, w1_ref, b1_ref, w2_ref, b2_ref, o_ref):
    j = pl.program_id(0)
    nb = pl.num_programs(0)

    @pl.when(j == 0)
    def _():
        o_ref[...] = jnp.zeros_like(o_ref)

    x = jnp.maximum(x_ref[...], 0.0)
    h = jnp.maximum(x @ w1_ref[...] + b1_ref[...], 0.0)
    o_ref[...] += h @ w2_ref[...]

    @pl.when(j == nb - 1)
    def _():
        logits = o_ref[...] + b2_ref[...]
        m = jnp.max(logits, axis=-1, keepdims=True)
        e = jnp.exp(logits - m)
        o_ref[...] = e / jnp.sum(e, axis=-1, keepdims=True)


def _mlp_head(x, w1, b1, w2, b2):
    bsz = x.shape[0]
    grid = (SIZE_OUT1 // _MLP_BLK,)
    return pl.pallas_call(
        _mlp_kernel,
        grid=grid,
        in_specs=[
            pl.BlockSpec((bsz, SIZE_IN1), lambda j: (0, 0)),
            pl.BlockSpec((SIZE_IN1, _MLP_BLK), lambda j: (0, j)),
            pl.BlockSpec((1, _MLP_BLK), lambda j: (0, j)),
            pl.BlockSpec((_MLP_BLK, OUTPUT_SIZE), lambda j: (j, 0)),
            pl.BlockSpec((1, OUTPUT_SIZE), lambda j: (0, 0)),
        ],
        out_specs=pl.BlockSpec((bsz, OUTPUT_SIZE), lambda j: (0, 0)),
        out_shape=jax.ShapeDtypeStruct((bsz, OUTPUT_SIZE), jnp.float32),
    )(x, w1, b1, w2, b2)


def kernel(x_temporal, edge_index, edge_weight, batch,
           W_xz, b_xz, W_hz, b_hz, W_xr, b_xr, W_hr, b_hr,
           W_xh, b_xh, W_hh, b_hh, lin1_W, lin1_b, lin2_W, lin2_b):
    row, col = edge_index[0], edge_index[1]
    deg = jnp.zeros((N,), jnp.float32).at[row].add(edge_weight)
    dis = jnp.where(deg > 0, lax.rsqrt(jnp.where(deg > 0, deg, 1.0)), 0.0)
    norm = -dis[row] * edge_weight * dis[col]

    pad = _EPAD - E
    rowp = jnp.concatenate([row, jnp.zeros((pad,), jnp.int32)]).reshape(
        _NW, _NCHUNK, _CHUNK)
    colp = jnp.concatenate([col, jnp.zeros((pad,), jnp.int32)]).reshape(
        _NW, _NCHUNK, _CHUNK)
    normp = jnp.concatenate([norm, jnp.zeros((pad,), jnp.float32)]).reshape(
        _NW, _NCHUNK, _CHUNK)

    CX = T * EMBED
    z32 = jnp.zeros((_NPAD, OUT_CH), jnp.float32)
    zX = jnp.zeros((_NPAD, CX // 2), jnp.float32)

    CH = CX // 2
    prop32_1 = _make_prop_sc(OUT_CH, 1)
    prop32_2 = _make_prop_sc(OUT_CH, 2)
    propX_1 = _make_prop_sc(CH, 1, chunk=64)
    propX_2 = _make_prop_sc(CH, 2, chunk=64)
    rowp64 = rowp.reshape(_NW, -1, 64)
    colp64 = colp.reshape(_NW, -1, 64)
    normp64 = normp.reshape(_NW, -1, 64)

    X_all = x_temporal.transpose(1, 0, 2).reshape(N, CX)
    t1h, t2h = [], []
    for XH in (X_all[:, :CH], X_all[:, CH:]):
        a0, a1 = propX_1(rowp64, colp64, normp64, XH, zX)
        b0, b1 = propX_2(rowp64, colp64, normp64, a0, a1, zX)
        t1h.append((a0 + a1)[:N])
        t2h.append(2.0 * (b0 + b1)[:N] - XH)
    T1 = jnp.concatenate(t1h, axis=1)
    T2 = jnp.concatenate(t2h, axis=1)
    F = jnp.concatenate(
        [x_temporal,
         T1.reshape(N, T, EMBED).transpose(1, 0, 2),
         T2.reshape(N, T, EMBED).transpose(1, 0, 2)],
        axis=2,
    ).reshape(T * N, 3 * EMBED)
    W_x = jnp.concatenate(
        [jnp.concatenate([W_xz[k], W_xr[k], W_xh[k]], axis=1) for k in range(K)],
        axis=0,
    )
    b_x = jnp.concatenate([b_xz, b_xr, b_xh]).reshape(1, 3 * OUT_CH)
    XC = _xc_matmul(F, W_x, b_x).reshape(T, N, 3 * OUT_CH)

    W_zr = jnp.stack(
        [jnp.concatenate([W_hz[k], W_hr[k]], axis=1) for k in range(K)]
    )
    b_zr = jnp.concatenate([b_hz, b_hr]).reshape(1, 2 * OUT_CH)
    b_h2 = b_hh.reshape(1, OUT_CH)

    def step(H, xc_t):
        q0, q1 = prop32_1(rowp, colp, normp, H, z32)
        qq0, qq1 = prop32_2(rowp, colp, normp, q0, q1, z32)
        Z, RH = _gates_zr(xc_t, H, q0, q1, qq0, qq1, W_zr, b_zr)
        r0, r1 = prop32_1(rowp, colp, normp, RH, z32)
        rr0, rr1 = prop32_2(rowp, colp, normp, r0, r1, z32)
        Hn = _gates_h(xc_t, RH, r0, r1, rr0, rr1, Z, H, W_hh, b_h2)
        return Hn, None

    H0 = jnp.zeros((N, OUT_CH), jnp.float32)
    H, _ = lax.scan(step, H0, XC)

    xb = H.reshape(NUM_NODES, SIZE_IN1)
    return _mlp_head(xb, lin1_W, lin1_b.reshape(1, -1), lin2_W,
                     lin2_b.reshape(1, -1))
